# Initial kernel scaffold; baseline (speedup 1.0000x reference)
import jax
import jax.numpy as jnp
from jax.experimental import pallas as pl


def kernel(x, W_qkv, b_qkv, W_o, b_o):
    raise NotImplementedError("write your pallas kernel here")


# speedup vs baseline: 1.3042x; 1.3042x over previous
import functools

import jax
import jax.numpy as jnp
from jax.experimental import pallas as pl
from jax.experimental.pallas import tpu as pltpu

_N = 2
_D, _H, _W = 8, 32, 32
_C = 256
_NWIN = 4
_P3 = _NWIN ** 3
_NW = _N * _P3
_d, _h, _w = _D // _NWIN, _H // _NWIN, _W // _NWIN
_W3 = _d * _h * _w
_QK = 256
_DIM = 256
_HEADS = 8
_CH = _QK // _HEADS
_TOPK = 4
_SCALE = _QK ** -0.5

_BW = 8


def _qkv_kernel(x_ref, w_ref, b_ref, qkv_ref, qw_ref, kw_ref):
    xb = x_ref[...].reshape(_BW * _W3, _C)
    y = jnp.dot(xb, w_ref[...], preferred_element_type=jnp.float32)
    y = y + b_ref[...]
    y3 = y.reshape(_BW, _W3, 2 * _QK + _DIM)
    qkv_ref[...] = y3
    inv = 1.0 / _W3
    qw_ref[...] = jnp.sum(y3[:, :, :_QK], axis=1) * inv
    kw_ref[...] = jnp.sum(y3[:, :, _QK:2 * _QK], axis=1) * inv


def _routing_kernel(qw_ref, kw_ref, idx_ref):
    iota = jax.lax.broadcasted_iota(jnp.int32, (_P3, _P3), 1)
    for b in range(_N):
        qs = qw_ref[b * _P3:(b + 1) * _P3, :] * _SCALE
        ks = kw_ref[b * _P3:(b + 1) * _P3, :]
        logits = jax.lax.dot_general(
            qs, ks, (((1,), (1,)), ((), ())),
            preferred_element_type=jnp.float32)
        cols = []
        for _ in range(_TOPK):
            m = jnp.max(logits, axis=-1, keepdims=True)
            sel = logits == m
            idx = jnp.min(jnp.where(sel, iota, _P3), axis=-1)
            cols.append(idx + b * _P3)
            logits = jnp.where(iota == idx[:, None], -jnp.inf, logits)
        idx_ref[b * _P3:(b + 1) * _P3, :] = jnp.concatenate(
            [c[:, None] for c in cols], axis=1)


def _attn_kernel(idx_ref, q_ref, k0, k1, k2, k3, v0, v1, v2, v3,
                 wo_ref, bo_ref, out_ref):
    del idx_ref
    q = q_ref[0]
    k_all = jnp.concatenate([k0[0], k1[0], k2[0], k3[0]], axis=0)
    v_all = jnp.concatenate([v0[0], v1[0], v2[0], v3[0]], axis=0)
    outs = []
    for hh in range(_HEADS):
        lo = hh * _CH
        qh = q[:, lo:lo + _CH] * _SCALE
        kh = k_all[:, lo:lo + _CH]
        logits = jax.lax.dot_general(
            qh, kh, (((1,), (1,)), ((), ())),
            preferred_element_type=jnp.float32)
        m = jnp.max(logits, axis=-1, keepdims=True)
        e = jnp.exp(logits - m)
        p = e / jnp.sum(e, axis=-1, keepdims=True)
        outs.append(jnp.dot(p, v_all[:, lo:lo + _CH],
                            preferred_element_type=jnp.float32))
    o = jnp.concatenate(outs, axis=-1)
    res = jnp.dot(o, wo_ref[...], preferred_element_type=jnp.float32)
    out_ref[0] = res + bo_ref[...]


@functools.partial(jax.jit, static_argnames=())
def kernel(x, W_qkv, b_qkv, W_o, b_o):
    n = _N
    xw = x.reshape(n, _NWIN, _d, _NWIN, _h, _NWIN, _w, _C)
    xw = jnp.transpose(xw, (0, 1, 3, 5, 2, 4, 6, 7)).reshape(_NW, _W3, _C)

    ckv = 2 * _QK + _DIM
    qkv, q_win, k_win = pl.pallas_call(
        _qkv_kernel,
        grid=(_NW // _BW,),
        in_specs=[
            pl.BlockSpec((_BW, _W3, _C), lambda g: (g, 0, 0)),
            pl.BlockSpec((_C, ckv), lambda g: (0, 0)),
            pl.BlockSpec((1, ckv), lambda g: (0, 0)),
        ],
        out_specs=[
            pl.BlockSpec((_BW, _W3, ckv), lambda g: (g, 0, 0)),
            pl.BlockSpec((_BW, _QK), lambda g: (g, 0)),
            pl.BlockSpec((_BW, _QK), lambda g: (g, 0)),
        ],
        out_shape=[
            jax.ShapeDtypeStruct((_NW, _W3, ckv), jnp.float32),
            jax.ShapeDtypeStruct((_NW, _QK), jnp.float32),
            jax.ShapeDtypeStruct((_NW, _QK), jnp.float32),
        ],
    )(xw, W_qkv, b_qkv.reshape(1, ckv))

    topk_idx = pl.pallas_call(
        _routing_kernel,
        out_shape=jax.ShapeDtypeStruct((_NW, _TOPK), jnp.int32),
    )(q_win, k_win)

    idx_flat = topk_idx.reshape(_NW * _TOPK)

    def q_map(g, idx):
        return (g, 0, 0)

    def k_map(t):
        def f(g, idx):
            return (idx[_TOPK * g + t], 0, 1)
        return f

    def v_map(t):
        def f(g, idx):
            return (idx[_TOPK * g + t], 0, 2)
        return f

    kv_spec = lambda f: pl.BlockSpec((1, _W3, _QK), f)
    out_win = pl.pallas_call(
        _attn_kernel,
        grid_spec=pltpu.PrefetchScalarGridSpec(
            num_scalar_prefetch=1,
            grid=(_NW,),
            in_specs=[
                pl.BlockSpec((1, _W3, _QK), q_map),
                kv_spec(k_map(0)), kv_spec(k_map(1)),
                kv_spec(k_map(2)), kv_spec(k_map(3)),
                kv_spec(v_map(0)), kv_spec(v_map(1)),
                kv_spec(v_map(2)), kv_spec(v_map(3)),
                pl.BlockSpec((_DIM, _DIM), lambda g, idx: (0, 0)),
                pl.BlockSpec((1, _DIM), lambda g, idx: (0, 0)),
            ],
            out_specs=pl.BlockSpec((1, _W3, _DIM), lambda g, idx: (g, 0, 0)),
        ),
        out_shape=jax.ShapeDtypeStruct((_NW, _W3, _DIM), jnp.float32),
    )(idx_flat, qkv, qkv, qkv, qkv, qkv, qkv, qkv, qkv, qkv,
      W_o, b_o.reshape(1, _DIM))

    out = out_win.reshape(n, _NWIN, _NWIN, _NWIN, _d, _h, _w, _DIM)
    out = jnp.transpose(out, (0, 1, 4, 2, 5, 3, 6, 7)).reshape(
        n, _D, _H, _W, _DIM)
    return out


# speedup vs baseline: 1.4128x; 1.0833x over previous
import functools

import jax
import jax.numpy as jnp
from jax.experimental import pallas as pl
from jax.experimental.pallas import tpu as pltpu

_N = 2
_D, _H, _W = 8, 32, 32
_C = 256
_NWIN = 4
_P3 = _NWIN ** 3
_NW = _N * _P3
_d, _h, _w = _D // _NWIN, _H // _NWIN, _W // _NWIN
_W3 = _d * _h * _w
_QK = 256
_DIM = 256
_HEADS = 8
_CH = _QK // _HEADS
_TOPK = 4
_SCALE = _QK ** -0.5

_BW = 8


def _qkv_kernel(x_ref, w_ref, b_ref, qkv_ref, qw_ref, kw_ref):
    xb = x_ref[...].reshape(_BW * _W3, _C)
    y = jnp.dot(xb, w_ref[...], preferred_element_type=jnp.float32)
    y = y + b_ref[...]
    y3 = y.reshape(_BW, _W3, 2 * _QK + _DIM)
    q_scaled = y3[:, :, :_QK] * _SCALE
    rest = y3[:, :, _QK:]
    qkv_ref[...] = jnp.concatenate(
        [q_scaled, rest], axis=-1).astype(jnp.bfloat16)
    inv = 1.0 / _W3
    qw_ref[...] = jnp.sum(y3[:, :, :_QK], axis=1) * inv
    kw_ref[...] = jnp.sum(y3[:, :, _QK:2 * _QK], axis=1) * inv


def _routing_kernel(qw_ref, kw_ref, idx_ref):
    iota = jax.lax.broadcasted_iota(jnp.int32, (_P3, _P3), 1)
    for b in range(_N):
        qs = qw_ref[b * _P3:(b + 1) * _P3, :] * _SCALE
        ks = kw_ref[b * _P3:(b + 1) * _P3, :]
        logits = jax.lax.dot_general(
            qs, ks, (((1,), (1,)), ((), ())),
            preferred_element_type=jnp.float32)
        cols = []
        for _ in range(_TOPK):
            m = jnp.max(logits, axis=-1, keepdims=True)
            sel = logits == m
            idx = jnp.min(jnp.where(sel, iota, _P3), axis=-1)
            cols.append(idx + b * _P3)
            logits = jnp.where(iota == idx[:, None], -jnp.inf, logits)
        idx_ref[b * _P3:(b + 1) * _P3, :] = jnp.concatenate(
            [c[:, None] for c in cols], axis=1)


def _attn_kernel(idx_ref, q_ref, k0, k1, k2, k3, v0, v1, v2, v3,
                 wo_ref, bo_ref, out_ref):
    del idx_ref
    q = q_ref[0]
    ks = (k0[0], k1[0], k2[0], k3[0])
    vs = (v0[0], v1[0], v2[0], v3[0])
    outs = []
    for hh in range(_HEADS):
        lo = hh * _CH
        qh = q[:, lo:lo + _CH]
        lts = [jax.lax.dot_general(
            qh, kt[:, lo:lo + _CH], (((1,), (1,)), ((), ())),
            preferred_element_type=jnp.float32) for kt in ks]
        m = jnp.max(jnp.maximum(jnp.maximum(lts[0], lts[1]),
                                jnp.maximum(lts[2], lts[3])),
                    axis=-1, keepdims=True)
        es = [jnp.exp(lt - m) for lt in lts]
        s = jnp.sum(es[0] + es[1] + es[2] + es[3], axis=-1, keepdims=True)
        pv = None
        for et, vt in zip(es, vs):
            contrib = jnp.dot(et.astype(jnp.bfloat16), vt[:, lo:lo + _CH],
                              preferred_element_type=jnp.float32)
            pv = contrib if pv is None else pv + contrib
        outs.append(pv * (1.0 / s))
    o = jnp.concatenate(outs, axis=-1).astype(jnp.bfloat16)
    res = jnp.dot(o, wo_ref[...], preferred_element_type=jnp.float32)
    out_ref[0] = res + bo_ref[...]


@functools.partial(jax.jit, static_argnames=())
def kernel(x, W_qkv, b_qkv, W_o, b_o):
    n = _N
    xw = x.reshape(n, _NWIN, _d, _NWIN, _h, _NWIN, _w, _C)
    xw = jnp.transpose(xw, (0, 1, 3, 5, 2, 4, 6, 7)).reshape(_NW, _W3, _C)

    ckv = 2 * _QK + _DIM
    qkv, q_win, k_win = pl.pallas_call(
        _qkv_kernel,
        grid=(_NW // _BW,),
        in_specs=[
            pl.BlockSpec((_BW, _W3, _C), lambda g: (g, 0, 0)),
            pl.BlockSpec((_C, ckv), lambda g: (0, 0)),
            pl.BlockSpec((1, ckv), lambda g: (0, 0)),
        ],
        out_specs=[
            pl.BlockSpec((_BW, _W3, ckv), lambda g: (g, 0, 0)),
            pl.BlockSpec((_BW, _QK), lambda g: (g, 0)),
            pl.BlockSpec((_BW, _QK), lambda g: (g, 0)),
        ],
        out_shape=[
            jax.ShapeDtypeStruct((_NW, _W3, ckv), jnp.bfloat16),
            jax.ShapeDtypeStruct((_NW, _QK), jnp.float32),
            jax.ShapeDtypeStruct((_NW, _QK), jnp.float32),
        ],
    )(xw, W_qkv, b_qkv.reshape(1, ckv))

    topk_idx = pl.pallas_call(
        _routing_kernel,
        out_shape=jax.ShapeDtypeStruct((_NW, _TOPK), jnp.int32),
    )(q_win, k_win)

    idx_flat = topk_idx.reshape(_NW * _TOPK)

    def q_map(g, idx):
        return (g, 0, 0)

    def k_map(t):
        def f(g, idx):
            return (idx[_TOPK * g + t], 0, 1)
        return f

    def v_map(t):
        def f(g, idx):
            return (idx[_TOPK * g + t], 0, 2)
        return f

    kv_spec = lambda f: pl.BlockSpec((1, _W3, _QK), f)
    out_win = pl.pallas_call(
        _attn_kernel,
        grid_spec=pltpu.PrefetchScalarGridSpec(
            num_scalar_prefetch=1,
            grid=(_NW,),
            in_specs=[
                pl.BlockSpec((1, _W3, _QK), q_map),
                kv_spec(k_map(0)), kv_spec(k_map(1)),
                kv_spec(k_map(2)), kv_spec(k_map(3)),
                kv_spec(v_map(0)), kv_spec(v_map(1)),
                kv_spec(v_map(2)), kv_spec(v_map(3)),
                pl.BlockSpec((_DIM, _DIM), lambda g, idx: (0, 0)),
                pl.BlockSpec((1, _DIM), lambda g, idx: (0, 0)),
            ],
            out_specs=pl.BlockSpec((1, _W3, _DIM), lambda g, idx: (g, 0, 0)),
        ),
        out_shape=jax.ShapeDtypeStruct((_NW, _W3, _DIM), jnp.float32),
    )(idx_flat, qkv, qkv, qkv, qkv, qkv, qkv, qkv, qkv, qkv,
      W_o.astype(jnp.bfloat16), b_o.reshape(1, _DIM))

    out = out_win.reshape(n, _NWIN, _NWIN, _NWIN, _d, _h, _w, _DIM)
    out = jnp.transpose(out, (0, 1, 4, 2, 5, 3, 6, 7)).reshape(
        n, _D, _H, _W, _DIM)
    return out


# speedup vs baseline: 1.6833x; 1.1914x over previous
import functools

import jax
import jax.numpy as jnp
from jax.experimental import pallas as pl
from jax.experimental.pallas import tpu as pltpu

_N = 2
_D, _H, _W = 8, 32, 32
_C = 256
_NWIN = 4
_P3 = _NWIN ** 3
_NW = _N * _P3
_d, _h, _w = _D // _NWIN, _H // _NWIN, _W // _NWIN
_W3 = _d * _h * _w
_QK = 256
_DIM = 256
_HEADS = 8
_CH = _QK // _HEADS
_TOPK = 4
_SCALE = _QK ** -0.5

_BW = 8
_BC = 2


def _qkv_kernel(x_ref, w_ref, b_ref, qs_ref, kv_ref, qw_ref, kw_ref):
    xb = x_ref[...].reshape(_BW * _W3, _C)
    y = jnp.dot(xb, w_ref[...], preferred_element_type=jnp.float32)
    y = y + b_ref[...]
    y3 = y.reshape(_BW, _W3, 2 * _QK + _DIM)
    qs_ref[...] = (y3[:, :, :_QK] * _SCALE).astype(jnp.bfloat16)
    kv_ref[...] = y3[:, :, _QK:].astype(jnp.bfloat16)
    inv = 1.0 / _W3
    qw_ref[...] = jnp.sum(y3[:, :, :_QK], axis=1) * inv
    kw_ref[...] = jnp.sum(y3[:, :, _QK:2 * _QK], axis=1) * inv


def _routing_kernel(qw_ref, kw_ref, idx_ref):
    iota = jax.lax.broadcasted_iota(jnp.int32, (_P3, _P3), 1)
    for b in range(_N):
        qs = qw_ref[b * _P3:(b + 1) * _P3, :] * _SCALE
        ks = kw_ref[b * _P3:(b + 1) * _P3, :]
        logits = jax.lax.dot_general(
            qs, ks, (((1,), (1,)), ((), ())),
            preferred_element_type=jnp.float32)
        cols = []
        for _ in range(_TOPK):
            m = jnp.max(logits, axis=-1, keepdims=True)
            sel = logits == m
            idx = jnp.min(jnp.where(sel, iota, _P3), axis=-1)
            cols.append(idx + b * _P3)
            logits = jnp.where(iota == idx[:, None], -jnp.inf, logits)
        idx_ref[b * _P3:(b + 1) * _P3, :] = jnp.concatenate(
            [c[:, None] for c in cols], axis=1)


def _attn_kernel(idx_ref, q_ref, kv0, kv1, kv2, kv3, kv4, kv5, kv6, kv7,
                 wo_ref, bo_ref, out_ref):
    del idx_ref
    kv_refs = (kv0, kv1, kv2, kv3, kv4, kv5, kv6, kv7)
    for j in range(_BC):
        q = q_ref[j]
        kvs = [kv_refs[_TOPK * j + t][0] for t in range(_TOPK)]
        o_parts = []
        for hh in range(_HEADS):
            lo = hh * _CH
            qh = q[:, lo:lo + _CH]
            lts = [jax.lax.dot_general(
                kt[:, lo:lo + _CH], qh, (((1,), (1,)), ((), ())),
                preferred_element_type=jnp.float32) for kt in kvs]
            cm = jnp.maximum(jnp.maximum(lts[0], lts[1]),
                             jnp.maximum(lts[2], lts[3]))
            m = jnp.max(cm, axis=0, keepdims=True)
            es = [jnp.exp(lt - m) for lt in lts]
            s = jnp.sum(es[0] + es[1] + es[2] + es[3],
                        axis=0, keepdims=True)
            pv = None
            for et, kt in zip(es, kvs):
                vh = kt[:, _QK + lo:_QK + lo + _CH]
                contrib = jax.lax.dot_general(
                    vh, et.astype(jnp.bfloat16), (((0,), (0,)), ((), ())),
                    preferred_element_type=jnp.float32)
                pv = contrib if pv is None else pv + contrib
            o_parts.append(pv * (1.0 / s))
        o_t = jnp.concatenate(o_parts, axis=0).astype(jnp.bfloat16)
        res_t = jax.lax.dot_general(
            wo_ref[...], o_t, (((0,), (0,)), ((), ())),
            preferred_element_type=jnp.float32)
        out_ref[j] = res_t + bo_ref[...]


@functools.partial(jax.jit, static_argnames=())
def kernel(x, W_qkv, b_qkv, W_o, b_o):
    n = _N
    xw = x.reshape(n, _NWIN, _d, _NWIN, _h, _NWIN, _w, _C)
    xw = jnp.transpose(xw, (0, 1, 3, 5, 2, 4, 6, 7)).reshape(_NW, _W3, _C)

    ckv = 2 * _QK + _DIM
    qs, kv, q_win, k_win = pl.pallas_call(
        _qkv_kernel,
        grid=(_NW // _BW,),
        in_specs=[
            pl.BlockSpec((_BW, _W3, _C), lambda g: (g, 0, 0)),
            pl.BlockSpec((_C, ckv), lambda g: (0, 0)),
            pl.BlockSpec((1, ckv), lambda g: (0, 0)),
        ],
        out_specs=[
            pl.BlockSpec((_BW, _W3, _QK), lambda g: (g, 0, 0)),
            pl.BlockSpec((_BW, _W3, 2 * _QK), lambda g: (g, 0, 0)),
            pl.BlockSpec((_BW, _QK), lambda g: (g, 0)),
            pl.BlockSpec((_BW, _QK), lambda g: (g, 0)),
        ],
        out_shape=[
            jax.ShapeDtypeStruct((_NW, _W3, _QK), jnp.bfloat16),
            jax.ShapeDtypeStruct((_NW, _W3, 2 * _QK), jnp.bfloat16),
            jax.ShapeDtypeStruct((_NW, _QK), jnp.float32),
            jax.ShapeDtypeStruct((_NW, _QK), jnp.float32),
        ],
    )(xw, W_qkv, b_qkv.reshape(1, ckv))

    topk_idx = pl.pallas_call(
        _routing_kernel,
        out_shape=jax.ShapeDtypeStruct((_NW, _TOPK), jnp.int32),
    )(q_win, k_win)

    idx_flat = topk_idx.reshape(_NW * _TOPK)

    def kv_map(t):
        def f(g, idx):
            return (idx[_TOPK * _BC * g + t], 0, 0)
        return f

    kv_spec = lambda f: pl.BlockSpec((1, _W3, 2 * _QK), f)
    out_win = pl.pallas_call(
        _attn_kernel,
        grid_spec=pltpu.PrefetchScalarGridSpec(
            num_scalar_prefetch=1,
            grid=(_NW // _BC,),
            in_specs=[
                pl.BlockSpec((_BC, _W3, _QK), lambda g, idx: (g, 0, 0)),
                kv_spec(kv_map(0)), kv_spec(kv_map(1)),
                kv_spec(kv_map(2)), kv_spec(kv_map(3)),
                kv_spec(kv_map(4)), kv_spec(kv_map(5)),
                kv_spec(kv_map(6)), kv_spec(kv_map(7)),
                pl.BlockSpec((_DIM, _DIM), lambda g, idx: (0, 0)),
                pl.BlockSpec((_DIM, 1), lambda g, idx: (0, 0)),
            ],
            out_specs=pl.BlockSpec((_BC, _DIM, _W3), lambda g, idx: (g, 0, 0)),
        ),
        out_shape=jax.ShapeDtypeStruct((_NW, _DIM, _W3), jnp.float32),
    )(idx_flat, qs, kv, kv, kv, kv, kv, kv, kv, kv,
      W_o.astype(jnp.bfloat16), b_o.reshape(_DIM, 1))

    out = out_win.reshape(n, _NWIN, _NWIN, _NWIN, _DIM, _d, _h, _w)
    out = jnp.transpose(out, (0, 1, 5, 2, 6, 3, 7, 4)).reshape(
        n, _D, _H, _W, _DIM)
    return out
